# baseline (speedup 1.0000x reference)
import jax
import jax.numpy as jnp
import numpy as np
from jax.experimental import pallas as pl
from jax.experimental.pallas import tpu as pltpu


def _att_bag_kernel(bag_labels_ref, xa0_ref, xa1_ref, xb0_ref, xb1_ref,
                    w_ref, b_ref, repre_ref, logits_ref):
    i = pl.program_id(0)
    C = w_ref.shape[0]
    H = w_ref.shape[1]

    def one_bag(lab, xh0, xh1):
        onehot = (jax.lax.broadcasted_iota(jnp.int32, (1, C), 1) == lab
                  ).astype(jnp.float32)
        w = jax.lax.dot_general(
            onehot, w_ref[...], (((1,), (0,)), ((), ())),
            preferred_element_type=jnp.float32,
        )
        l0 = jax.lax.dot_general(
            xh0, w, (((1,), (1,)), ((), ())),
            preferred_element_type=jnp.float32,
        )
        l1 = jax.lax.dot_general(
            xh1, w, (((1,), (1,)), ((), ())),
            preferred_element_type=jnp.float32,
        )
        m = jnp.maximum(jnp.max(l0), jnp.max(l1))
        p0 = jnp.exp(l0 - m)
        p1 = jnp.exp(l1 - m)
        s = jnp.sum(p0) + jnp.sum(p1)
        acc = jax.lax.dot_general(
            p0, xh0, (((0,), (0,)), ((), ())),
            preferred_element_type=jnp.float32,
        ) + jax.lax.dot_general(
            p1, xh1, (((0,), (0,)), ((), ())),
            preferred_element_type=jnp.float32,
        )
        repre = acc * (1.0 / s)
        row = jax.lax.dot_general(
            repre, w_ref[...], (((1,), (1,)), ((), ())),
            preferred_element_type=jnp.float32,
        ) + b_ref[...]
        return repre, row

    ra, rowa = one_bag(bag_labels_ref[2 * i], xa0_ref[...], xa1_ref[...])
    rb, rowb = one_bag(bag_labels_ref[2 * i + 1], xb0_ref[...], xb1_ref[...])
    repre_ref[...] = jnp.concatenate([ra, rb], axis=0).reshape(2, 1, H)
    logits_ref[...] = jnp.concatenate([rowa, rowb], axis=0).reshape(2, 1, C)


def kernel(x, labels, scopes, W, b):
    N, H = x.shape
    C = W.shape[0]
    B = scopes.shape[0]
    L = N // B
    Lh = L // 2

    starts = jnp.asarray(scopes)[:, 0].astype(jnp.int32)
    bag_labels = jnp.take(labels, starts, axis=0).astype(jnp.int32)
    b2 = b.reshape(1, C)

    def _xspec(k):
        return pl.BlockSpec((Lh, H), lambda i, *_: (4 * i + k, 0))

    grid_spec = pltpu.PrefetchScalarGridSpec(
        num_scalar_prefetch=1,
        grid=(B // 2,),
        in_specs=[_xspec(k) for k in range(4)] + [
            pl.BlockSpec((C, H), lambda i, *_: (0, 0)),
            pl.BlockSpec((1, C), lambda i, *_: (0, 0)),
        ],
        out_specs=[
            pl.BlockSpec((2, 1, H), lambda i, *_: (i, 0, 0)),
            pl.BlockSpec((2, 1, C), lambda i, *_: (i, 0, 0)),
        ],
    )
    repre3, logits3 = pl.pallas_call(
        _att_bag_kernel,
        grid_spec=grid_spec,
        out_shape=[
            jax.ShapeDtypeStruct((B, 1, H), jnp.float32),
            jax.ShapeDtypeStruct((B, 1, C), jnp.float32),
        ],
        compiler_params=pltpu.CompilerParams(
            dimension_semantics=("parallel",)
        ),
    )(bag_labels, x, x, x, x, W, b2)
    return (repre3.reshape(B, H), logits3.reshape(B, C))


# speedup vs baseline: 1.0013x; 1.0013x over previous
import jax
import jax.numpy as jnp
from jax.experimental import pallas as pl
from jax.experimental.pallas import tpu as pltpu


def _att_bag_kernel(bag_labels_ref, xa_ref, xb_ref, w_ref, b_ref,
                    repre_ref, logits_ref):
    i = pl.program_id(0)
    C = w_ref.shape[0]
    H = w_ref.shape[1]

    def one_bag(lab, x):
        onehot = (jax.lax.broadcasted_iota(jnp.int32, (1, C), 1) == lab
                  ).astype(jnp.float32)
        w = jax.lax.dot_general(
            onehot, w_ref[...], (((1,), (0,)), ((), ())),
            preferred_element_type=jnp.float32,
        )
        logit = jax.lax.dot_general(
            x, w, (((1,), (1,)), ((), ())), preferred_element_type=jnp.float32
        )
        m = jnp.max(logit)
        p = jnp.exp(logit - m)
        s = jnp.sum(p)
        acc = jax.lax.dot_general(
            p, x, (((0,), (0,)), ((), ())), preferred_element_type=jnp.float32
        )
        repre = acc * (1.0 / s)
        row = jax.lax.dot_general(
            repre, w_ref[...], (((1,), (1,)), ((), ())),
            preferred_element_type=jnp.float32,
        ) + b_ref[...]
        return repre, row

    ra, rowa = one_bag(bag_labels_ref[2 * i], xa_ref[...])
    rb, rowb = one_bag(bag_labels_ref[2 * i + 1], xb_ref[...])
    repre_ref[...] = jnp.concatenate([ra, rb], axis=0).reshape(2, 1, H)
    logits_ref[...] = jnp.concatenate([rowa, rowb], axis=0).reshape(2, 1, C)


def kernel(x, labels, scopes, W, b):
    N, H = x.shape
    C = W.shape[0]
    B = scopes.shape[0]
    L = N // B

    starts = jnp.asarray(scopes)[:, 0].astype(jnp.int32)
    bag_labels = jnp.take(labels, starts, axis=0).astype(jnp.int32)
    b2 = b.reshape(1, C)

    grid_spec = pltpu.PrefetchScalarGridSpec(
        num_scalar_prefetch=1,
        grid=(B // 2,),
        in_specs=[
            pl.BlockSpec((L, H), lambda i, *_: (2 * i, 0)),
            pl.BlockSpec((L, H), lambda i, *_: (2 * i + 1, 0)),
            pl.BlockSpec((C, H), lambda i, *_: (0, 0)),
            pl.BlockSpec((1, C), lambda i, *_: (0, 0)),
        ],
        out_specs=[
            pl.BlockSpec((2, 1, H), lambda i, *_: (i, 0, 0)),
            pl.BlockSpec((2, 1, C), lambda i, *_: (i, 0, 0)),
        ],
    )
    repre3, logits3 = pl.pallas_call(
        _att_bag_kernel,
        grid_spec=grid_spec,
        out_shape=[
            jax.ShapeDtypeStruct((B, 1, H), jnp.float32),
            jax.ShapeDtypeStruct((B, 1, C), jnp.float32),
        ],
        compiler_params=pltpu.CompilerParams(
            dimension_semantics=("parallel",)
        ),
    )(bag_labels, x, x, W, b2)
    return (repre3.reshape(B, H), logits3.reshape(B, C))
